# Initial kernel scaffold; baseline (speedup 1.0000x reference)
import jax
import jax.numpy as jnp
from jax.experimental import pallas as pl


def kernel(x, W, b):
    raise NotImplementedError("write your pallas kernel here")


# speedup vs baseline: 17.3872x; 17.3872x over previous
import functools

import jax
import jax.numpy as jnp
from jax.experimental import pallas as pl


def _median_body(x_ref, o_ref, *, k):
    xi = x_ref[...]
    bits = jax.lax.bitcast_convert_type(xi, jnp.int32)
    flip = jnp.int32(0x7FFFFFFF)
    s = jnp.where(bits >= 0, bits, jnp.bitwise_xor(bits, flip))

    rows = xi.shape[0]
    lo0 = jnp.full((rows, 1), jnp.iinfo(jnp.int32).min, jnp.int32)
    hi0 = jnp.full((rows, 1), jnp.iinfo(jnp.int32).max, jnp.int32)

    def step(_, carry):
        lo, hi = carry
        mid = (lo >> 1) + (hi >> 1) + (lo & hi & 1)
        cnt = jnp.sum((s > mid).astype(jnp.int32), axis=1, keepdims=True)
        ge = cnt >= k
        return jnp.where(ge, mid + 1, lo), jnp.where(ge, hi, mid)

    lo, hi = jax.lax.fori_loop(0, 32, step, (lo0, hi0))
    a = lo
    bpat = jnp.where(a >= 0, a, jnp.bitwise_xor(a, flip))
    med = jax.lax.bitcast_convert_type(bpat, jnp.float32)
    o_ref[...] = jnp.broadcast_to(med, o_ref.shape)


def _dense_body(m_ref, w_ref, b_ref, o_ref):
    o_ref[...] = (
        jnp.dot(m_ref[...], w_ref[...], preferred_element_type=jnp.float32)
        + b_ref[...]
    )


def kernel(x, W, b):
    B, C, H, Wsp = x.shape
    N = H * Wsp
    k = N // 2
    rows = B * C
    xf = x.reshape(rows, N)

    R = 8
    med = pl.pallas_call(
        functools.partial(_median_body, k=k),
        grid=(rows // R,),
        in_specs=[pl.BlockSpec((R, N), lambda i: (i, 0))],
        out_specs=pl.BlockSpec((R, 128), lambda i: (i, 0)),
        out_shape=jax.ShapeDtypeStruct((rows, 128), jnp.float32),
    )(xf)

    med = med[:, 0].reshape(B, C)
    out = pl.pallas_call(
        _dense_body,
        out_shape=jax.ShapeDtypeStruct((B, W.shape[0]), jnp.float32),
    )(med, W.T, b.reshape(1, -1))
    return out
